# Initial kernel scaffold; baseline (speedup 1.0000x reference)
import jax
import jax.numpy as jnp
from jax.experimental import pallas as pl


def kernel(t, pos, edge_index, W1, b1, W2, b2, W3, b3):
    raise NotImplementedError("write your pallas kernel here")


# speedup vs baseline: 14.2587x; 14.2587x over previous
import functools

import jax
import jax.numpy as jnp
from jax import lax
from jax.experimental import pallas as pl
from jax.experimental.pallas import tpu as pltpu
from jax.experimental.pallas import tpu_sc as plsc

N = 50000
E = 1600000

NC = 2
NS = 16
NW = NC * NS

E_PAD = 1638400
EW = E_PAD // NW
C = 2048
NWIN = EW // C
G = C // 16

N_PAD = 50176
RT = N_PAD // NS
RW = N_PAD // NW
R_LAST = N - (NW - 1) * RW

BC = 32768

_MESH = plsc.VectorSubcoreMesh(core_axis_name="c", subcore_axis_name="s")
_SC_PARAMS = pltpu.CompilerParams(needs_layout_passes=False, use_tc_tiling_on_sc=False)


def _worker_id():
    return lax.axis_index("s") * NC + lax.axis_index("c")


def _gather_body(nf, ei, ej, feat, idxi, idxj, rowsi, rowsj,
                 bdt, br2, bdx, bdy, bdz, sem):
    w = _worker_id()
    base = w * EW
    iota = lax.iota(jnp.int32, 16)
    c0 = jnp.zeros((16,), jnp.int32)
    c1 = c0 + 1
    c2 = c0 + 2
    c3 = c0 + 3

    def win_body(win, carry):
        off = base + win * C
        pltpu.sync_copy(ei.at[pl.ds(off, C)], idxi)
        pltpu.sync_copy(ej.at[pl.ds(off, C)], idxj)
        cp1 = pltpu.async_copy(nf.at[idxi], rowsi, sem)
        cp2 = pltpu.async_copy(nf.at[idxj], rowsj, sem)
        cp1.wait()
        cp2.wait()

        def grp(g, carry2):
            rows = g * 16 + iota
            xi = plsc.load_gather(rowsi, [rows, c0])
            yi = plsc.load_gather(rowsi, [rows, c1])
            zi = plsc.load_gather(rowsi, [rows, c2])
            ti = plsc.load_gather(rowsi, [rows, c3])
            xj = plsc.load_gather(rowsj, [rows, c0])
            yj = plsc.load_gather(rowsj, [rows, c1])
            zj = plsc.load_gather(rowsj, [rows, c2])
            tj = plsc.load_gather(rowsj, [rows, c3])
            dx = xi - xj
            dy = yi - yj
            dz = zi - zj
            dt = ti - tj
            r2 = dx * dx + dy * dy + dz * dz
            sl = pl.ds(g * 16, 16)
            bdt[sl] = dt
            br2[sl] = r2
            bdx[sl] = dx
            bdy[sl] = dy
            bdz[sl] = dz
            return carry2

        lax.fori_loop(0, G, grp, 0)
        pltpu.sync_copy(bdt, feat.at[0, pl.ds(off, C)])
        pltpu.sync_copy(br2, feat.at[1, pl.ds(off, C)])
        pltpu.sync_copy(bdx, feat.at[2, pl.ds(off, C)])
        pltpu.sync_copy(bdy, feat.at[3, pl.ds(off, C)])
        pltpu.sync_copy(bdz, feat.at[4, pl.ds(off, C)])
        return carry

    lax.fori_loop(0, NWIN, win_body, 0)


_gather = functools.partial(
    pl.kernel,
    mesh=_MESH,
    out_type=jax.ShapeDtypeStruct((8, E_PAD), jnp.float32),
    scratch_types=[
        pltpu.VMEM((C,), jnp.int32),
        pltpu.VMEM((C,), jnp.int32),
        pltpu.VMEM((C, 8), jnp.float32),
        pltpu.VMEM((C, 8), jnp.float32),
        pltpu.VMEM((C,), jnp.float32),
        pltpu.VMEM((C,), jnp.float32),
        pltpu.VMEM((C,), jnp.float32),
        pltpu.VMEM((C,), jnp.float32),
        pltpu.VMEM((C,), jnp.float32),
        pltpu.SemaphoreType.DMA,
    ],
    compiler_params=_SC_PARAMS,
)(_gather_body)


def _mlp_body(feat_ref, w1_ref, b1_ref, w2_ref, b2_ref, w3_ref, b3_ref,
              out_ref):
    dn = (((1,), (0,)), ((), ()))
    prec = lax.Precision.DEFAULT
    x = feat_ref[0:2, :]
    h = lax.dot_general(w1_ref[...], x, dn, precision=prec,
                        preferred_element_type=jnp.float32)
    h = jnp.maximum(h + b1_ref[...][:, None], 0.0)
    h = lax.dot_general(w2_ref[...], h, dn, precision=prec,
                        preferred_element_type=jnp.float32)
    h = jnp.maximum(h + b2_ref[...][:, None], 0.0)
    wg = lax.dot_general(w3_ref[...], h, dn, precision=prec,
                         preferred_element_type=jnp.float32)
    wg = wg + b3_ref[...][:, None]
    r2 = feat_ref[1:2, :]
    rn = lax.rsqrt(jnp.maximum(r2, 1e-24))
    out_ref[...] = feat_ref[2:5, :] * (wg * rn)


def _mlp(feat, w1, b1, w2, b2, w3, b3):
    return pl.pallas_call(
        _mlp_body,
        grid=(E_PAD // BC,),
        in_specs=[
            pl.BlockSpec((8, BC), lambda i: (0, i)),
            pl.BlockSpec((20, 2), lambda i: (0, 0)),
            pl.BlockSpec((20,), lambda i: (0,)),
            pl.BlockSpec((20, 20), lambda i: (0, 0)),
            pl.BlockSpec((20,), lambda i: (0,)),
            pl.BlockSpec((1, 20), lambda i: (0, 0)),
            pl.BlockSpec((1,), lambda i: (0,)),
        ],
        out_specs=pl.BlockSpec((3, BC), lambda i: (0, i)),
        out_shape=jax.ShapeDtypeStruct((3, E_PAD), jnp.float32),
    )(feat, w1, b1, w2, b2, w3, b3)


def _scatter_body(vals, ei, parts, idxv, bx, by, bz, aos, zb, acc):
    c = lax.axis_index("c")
    s = lax.axis_index("s")
    w = _worker_id()
    iota = lax.iota(jnp.int32, 16)
    zeros16 = jnp.zeros((16,), jnp.float32)
    c0 = jnp.zeros((16,), jnp.int32)
    c1 = c0 + 1
    c2 = c0 + 2
    c3 = c0 + 3

    def zfill(k, carry):
        o = k * 16 + iota
        plsc.store_scatter(zb, [lax.shift_right_logical(o, 3),
                                lax.bitwise_and(o, 7)], zeros16)
        return carry

    lax.fori_loop(0, RT * 8 // 16, zfill, 0)
    pltpu.sync_copy(zb, acc.at[pl.ds(s * RT, RT)])

    def z3(k, carry):
        o = k * 16 + iota
        plsc.store_scatter(aos, [lax.shift_right_logical(o, 3),
                                 lax.bitwise_and(o, 7)], zeros16)
        return carry

    lax.fori_loop(0, C * 8 // 16, z3, 0)
    plsc.subcore_barrier()

    base = w * EW

    def win_body(win, carry):
        off = base + win * C
        pltpu.sync_copy(ei.at[pl.ds(off, C)], idxv)
        pltpu.sync_copy(vals.at[0, pl.ds(off, C)], bx)
        pltpu.sync_copy(vals.at[1, pl.ds(off, C)], by)
        pltpu.sync_copy(vals.at[2, pl.ds(off, C)], bz)

        def grp(g, carry2):
            rows = g * 16 + iota
            sl = pl.ds(g * 16, 16)
            plsc.store_scatter(aos, [rows, c0], bx[sl])
            plsc.store_scatter(aos, [rows, c1], by[sl])
            plsc.store_scatter(aos, [rows, c2], bz[sl])
            return carry2

        lax.fori_loop(0, G, grp, 0)
        pltpu.sync_copy(aos, acc.at[idxv], add=True)
        return carry

    lax.fori_loop(0, NWIN, win_body, 0)
    plsc.subcore_barrier()
    pltpu.sync_copy(acc.at[pl.ds(s * RT, RT)],
                    parts.at[c, pl.ds(s * RT, RT), :])


_scatter = functools.partial(
    pl.kernel,
    mesh=_MESH,
    out_type=jax.ShapeDtypeStruct((NC, N_PAD, 8), jnp.float32),
    scratch_types=[
        pltpu.VMEM((C,), jnp.int32),
        pltpu.VMEM((C,), jnp.float32),
        pltpu.VMEM((C,), jnp.float32),
        pltpu.VMEM((C,), jnp.float32),
        pltpu.VMEM((C, 8), jnp.float32),
        pltpu.VMEM((RT, 8), jnp.float32),
        pltpu.VMEM_SHARED((N_PAD, 8), jnp.float32),
    ],
    compiler_params=_SC_PARAMS,
)(_scatter_body)


def _combine_body(parts, out, b0, b1, bo):
    w = _worker_id()
    r0 = w * RW
    iota = lax.iota(jnp.int32, 16)
    pltpu.sync_copy(parts.at[0, pl.ds(r0, RW), :], b0)
    pltpu.sync_copy(parts.at[1, pl.ds(r0, RW), :], b1)

    def grp(k, carry):
        o = k * 16 + iota
        rows = lax.div(o, 3)
        cols = o - rows * 3
        v0 = plsc.load_gather(b0, [rows, cols])
        v1 = plsc.load_gather(b1, [rows, cols])
        plsc.store_scatter(bo, [rows, cols], v0 + v1)
        return carry

    lax.fori_loop(0, RW * 3 // 16, grp, 0)

    @pl.when(w != NW - 1)
    def _():
        pltpu.sync_copy(bo, out.at[pl.ds(r0, RW), :])

    @pl.when(w == NW - 1)
    def _():
        pltpu.sync_copy(bo.at[pl.ds(0, R_LAST), :],
                        out.at[pl.ds((NW - 1) * RW, R_LAST), :])


_combine = functools.partial(
    pl.kernel,
    mesh=_MESH,
    out_type=jax.ShapeDtypeStruct((N, 3), jnp.float32),
    scratch_types=[
        pltpu.VMEM((RW, 8), jnp.float32),
        pltpu.VMEM((RW, 8), jnp.float32),
        pltpu.VMEM((RW, 3), jnp.float32),
    ],
    compiler_params=_SC_PARAMS,
)(_combine_body)


def kernel(t, pos, edge_index, W1, b1, W2, b2, W3, b3):
    nf = jnp.concatenate([pos, t, jnp.zeros((N, 4), jnp.float32)],
                         axis=1)
    pad = jnp.zeros((E_PAD - E,), edge_index.dtype)
    ei = jnp.concatenate([edge_index[0], pad])
    ej = jnp.concatenate([edge_index[1], pad])
    feat = _gather(nf, ei, ej)
    vals = _mlp(feat, W1, b1, W2, b2, W3, b3)
    parts = _scatter(vals, ei)
    return _combine(parts)


# speedup vs baseline: 16.4884x; 1.1564x over previous
import functools

import jax
import jax.numpy as jnp
from jax import lax
from jax.experimental import pallas as pl
from jax.experimental.pallas import tpu as pltpu
from jax.experimental.pallas import tpu_sc as plsc

N = 50000
E = 1600000

NC = 2
NS = 16
NW = NC * NS

E_PAD = 1638400
EW = E_PAD // NW
C = 2048
NWIN = EW // C
G = C // 16

N_PAD = 50176
RT = N_PAD // NS
RW = N_PAD // NW
R_LAST = N - (NW - 1) * RW

BC = 32768

_MESH = plsc.VectorSubcoreMesh(core_axis_name="c", subcore_axis_name="s")
_SC_PARAMS = pltpu.CompilerParams(needs_layout_passes=False, use_tc_tiling_on_sc=False)


def _worker_id():
    return lax.axis_index("s") * NC + lax.axis_index("c")


def _gather_body(nf, ei, ej, feat, idxi, idxj, rowsi, rowsj,
                 bdt, br2, bdx, bdy, bdz, sem):
    w = _worker_id()
    base = w * EW
    iota = lax.iota(jnp.int32, 16)
    c0 = jnp.zeros((16,), jnp.int32)
    c1 = c0 + 1
    c2 = c0 + 2
    c3 = c0 + 3

    def win_body(win, carry):
        off = base + win * C
        pltpu.sync_copy(ei.at[pl.ds(off, C)], idxi)
        pltpu.sync_copy(ej.at[pl.ds(off, C)], idxj)
        cp1 = pltpu.async_copy(nf.at[idxi], rowsi, sem)
        cp2 = pltpu.async_copy(nf.at[idxj], rowsj, sem)
        cp1.wait()
        cp2.wait()

        def grp(g, carry2):
            rows = g * 16 + iota
            xi = plsc.load_gather(rowsi, [rows, c0])
            yi = plsc.load_gather(rowsi, [rows, c1])
            zi = plsc.load_gather(rowsi, [rows, c2])
            ti = plsc.load_gather(rowsi, [rows, c3])
            xj = plsc.load_gather(rowsj, [rows, c0])
            yj = plsc.load_gather(rowsj, [rows, c1])
            zj = plsc.load_gather(rowsj, [rows, c2])
            tj = plsc.load_gather(rowsj, [rows, c3])
            dx = xi - xj
            dy = yi - yj
            dz = zi - zj
            dt = ti - tj
            r2 = dx * dx + dy * dy + dz * dz
            sl = pl.ds(g * 16, 16)
            bdt[sl] = dt
            br2[sl] = r2
            bdx[sl] = dx
            bdy[sl] = dy
            bdz[sl] = dz
            return carry2

        lax.fori_loop(0, G, grp, 0)
        pltpu.sync_copy(bdt, feat.at[0, pl.ds(off, C)])
        pltpu.sync_copy(br2, feat.at[1, pl.ds(off, C)])
        pltpu.sync_copy(bdx, feat.at[2, pl.ds(off, C)])
        pltpu.sync_copy(bdy, feat.at[3, pl.ds(off, C)])
        pltpu.sync_copy(bdz, feat.at[4, pl.ds(off, C)])
        return carry

    lax.fori_loop(0, NWIN, win_body, 0)


_gather = functools.partial(
    pl.kernel,
    mesh=_MESH,
    out_type=jax.ShapeDtypeStruct((8, E_PAD), jnp.float32),
    scratch_types=[
        pltpu.VMEM((C,), jnp.int32),
        pltpu.VMEM((C,), jnp.int32),
        pltpu.VMEM((C, 8), jnp.float32),
        pltpu.VMEM((C, 8), jnp.float32),
        pltpu.VMEM((C,), jnp.float32),
        pltpu.VMEM((C,), jnp.float32),
        pltpu.VMEM((C,), jnp.float32),
        pltpu.VMEM((C,), jnp.float32),
        pltpu.VMEM((C,), jnp.float32),
        pltpu.SemaphoreType.DMA,
    ],
    compiler_params=_SC_PARAMS,
)(_gather_body)


def _mlp_body(feat_ref, w1_ref, b1_ref, w2_ref, b2_ref, w3_ref, b3_ref,
              out_ref):
    dn = (((1,), (0,)), ((), ()))
    prec = lax.Precision.DEFAULT
    x = feat_ref[0:2, :]
    h = lax.dot_general(w1_ref[...], x, dn, precision=prec,
                        preferred_element_type=jnp.float32)
    h = jnp.maximum(h + b1_ref[...][:, None], 0.0)
    h = lax.dot_general(w2_ref[...], h, dn, precision=prec,
                        preferred_element_type=jnp.float32)
    h = jnp.maximum(h + b2_ref[...][:, None], 0.0)
    wg = lax.dot_general(w3_ref[...], h, dn, precision=prec,
                         preferred_element_type=jnp.float32)
    wg = wg + b3_ref[...][:, None]
    r2 = feat_ref[1:2, :]
    rn = lax.rsqrt(jnp.maximum(r2, 1e-24))
    out_ref[...] = feat_ref[2:5, :] * (wg * rn)


def _mlp(feat, w1, b1, w2, b2, w3, b3):
    return pl.pallas_call(
        _mlp_body,
        grid=(E_PAD // BC,),
        in_specs=[
            pl.BlockSpec((8, BC), lambda i: (0, i)),
            pl.BlockSpec((20, 2), lambda i: (0, 0)),
            pl.BlockSpec((20,), lambda i: (0,)),
            pl.BlockSpec((20, 20), lambda i: (0, 0)),
            pl.BlockSpec((20,), lambda i: (0,)),
            pl.BlockSpec((1, 20), lambda i: (0, 0)),
            pl.BlockSpec((1,), lambda i: (0,)),
        ],
        out_specs=pl.BlockSpec((3, BC), lambda i: (0, i)),
        out_shape=jax.ShapeDtypeStruct((3, E_PAD), jnp.float32),
    )(feat, w1, b1, w2, b2, w3, b3)


def _scatter_body(vals, ei, parts, idxv, bx, by, bz, aos, zb, acc):
    c = lax.axis_index("c")
    s = lax.axis_index("s")
    w = _worker_id()
    iota = lax.iota(jnp.int32, 16)
    zeros16 = jnp.zeros((16,), jnp.float32)
    c0 = jnp.zeros((16,), jnp.int32)
    c1 = c0 + 1
    c2 = c0 + 2
    c3 = c0 + 3

    def zfill(k, carry):
        o = k * 16 + iota
        plsc.store_scatter(zb, [lax.shift_right_logical(o, 3),
                                lax.bitwise_and(o, 7)], zeros16)
        return carry

    lax.fori_loop(0, RT * 8 // 16, zfill, 0)
    pltpu.sync_copy(zb, acc.at[pl.ds(s * RT, RT)])

    def z3(k, carry):
        o = k * 16 + iota
        plsc.store_scatter(aos, [lax.shift_right_logical(o, 3),
                                 lax.bitwise_and(o, 7)], zeros16)
        return carry

    lax.fori_loop(0, C * 8 // 16, z3, 0)
    plsc.subcore_barrier()

    base = w * EW

    def win_body(win, carry):
        off = base + win * C
        pltpu.sync_copy(ei.at[pl.ds(off, C)], idxv)
        pltpu.sync_copy(vals.at[0, pl.ds(off, C)], bx)
        pltpu.sync_copy(vals.at[1, pl.ds(off, C)], by)
        pltpu.sync_copy(vals.at[2, pl.ds(off, C)], bz)

        def grp(g, carry2):
            rows = g * 16 + iota
            sl = pl.ds(g * 16, 16)
            plsc.store_scatter(aos, [rows, c0], bx[sl])
            plsc.store_scatter(aos, [rows, c1], by[sl])
            plsc.store_scatter(aos, [rows, c2], bz[sl])
            return carry2

        lax.fori_loop(0, G, grp, 0)
        pltpu.sync_copy(aos, acc.at[idxv], add=True)
        return carry

    lax.fori_loop(0, NWIN, win_body, 0)
    plsc.subcore_barrier()
    pltpu.sync_copy(acc.at[pl.ds(s * RT, RT)],
                    parts.at[c, pl.ds(s * RT, RT), :])


_scatter = functools.partial(
    pl.kernel,
    mesh=_MESH,
    out_type=jax.ShapeDtypeStruct((NC, N_PAD, 8), jnp.float32),
    scratch_types=[
        pltpu.VMEM((C,), jnp.int32),
        pltpu.VMEM((C,), jnp.float32),
        pltpu.VMEM((C,), jnp.float32),
        pltpu.VMEM((C,), jnp.float32),
        pltpu.VMEM((C, 8), jnp.float32),
        pltpu.VMEM((RT, 8), jnp.float32),
        pltpu.VMEM_SHARED((N_PAD, 8), jnp.float32),
    ],
    compiler_params=_SC_PARAMS,
)(_scatter_body)


def _combine_body(parts, out, b0, b1, bo):
    w = _worker_id()
    r0 = w * RW
    iota = lax.iota(jnp.int32, 16)
    pltpu.sync_copy(parts.at[0, pl.ds(r0, RW), :], b0)
    pltpu.sync_copy(parts.at[1, pl.ds(r0, RW), :], b1)

    def grp(k, carry):
        o = k * 16 + iota
        rows = lax.div(o, 3)
        cols = o - rows * 3
        v0 = plsc.load_gather(b0, [rows, cols])
        v1 = plsc.load_gather(b1, [rows, cols])
        plsc.store_scatter(bo, [rows, cols], v0 + v1)
        return carry

    lax.fori_loop(0, RW * 3 // 16, grp, 0)

    @pl.when(w != NW - 1)
    def _():
        pltpu.sync_copy(bo, out.at[pl.ds(r0, RW), :])

    @pl.when(w == NW - 1)
    def _():
        pltpu.sync_copy(bo.at[pl.ds(0, R_LAST), :],
                        out.at[pl.ds((NW - 1) * RW, R_LAST), :])


_combine = functools.partial(
    pl.kernel,
    mesh=_MESH,
    out_type=jax.ShapeDtypeStruct((N, 3), jnp.float32),
    scratch_types=[
        pltpu.VMEM((RW, 8), jnp.float32),
        pltpu.VMEM((RW, 8), jnp.float32),
        pltpu.VMEM((RW, 3), jnp.float32),
    ],
    compiler_params=_SC_PARAMS,
)(_combine_body)


def kernel(t, pos, edge_index, W1, b1, W2, b2, W3, b3):
    nf = jnp.concatenate([pos, t, jnp.zeros((N, 4), jnp.float32)],
                         axis=1)
    pad = jnp.arange(E_PAD - E, dtype=edge_index.dtype) % N
    ei = jnp.concatenate([edge_index[0], pad])
    ej = jnp.concatenate([edge_index[1], pad])
    feat = _gather(nf, ei, ej)
    vals = _mlp(feat, W1, b1, W2, b2, W3, b3)
    parts = _scatter(vals, ei)
    return _combine(parts)


# speedup vs baseline: 30.6467x; 1.8587x over previous
import functools

import jax
import jax.numpy as jnp
from jax import lax
from jax.experimental import pallas as pl
from jax.experimental.pallas import tpu as pltpu
from jax.experimental.pallas import tpu_sc as plsc

N = 50000
E = 1600000

NC = 2
NS = 16
NW = NC * NS

E_PAD = 1638400
EW = E_PAD // NW
C = 2048
NWIN = EW // C
G = C // 16

N_PAD = 50176
RT = N_PAD // NS
RW = N_PAD // NW
R_LAST = N - (NW - 1) * RW

BC = 32768

_MESH = plsc.VectorSubcoreMesh(core_axis_name="c", subcore_axis_name="s")
_SC_PARAMS = pltpu.CompilerParams(needs_layout_passes=False, use_tc_tiling_on_sc=False)


def _worker_id():
    return lax.axis_index("s") * NC + lax.axis_index("c")


def _gather_body(nf, ei, ej, feat, idxi, idxj, rowsi, rowsj,
                 bdt, bdx, bdy, bdz, sem):
    w = _worker_id()
    base = w * EW
    iota = lax.iota(jnp.int32, 16)
    c0 = jnp.zeros((16,), jnp.int32)
    c1 = c0 + 1
    c2 = c0 + 2
    c3 = c0 + 3

    def win_body(win, carry):
        off = base + win * C
        pltpu.sync_copy(ei.at[pl.ds(off, C)], idxi)
        pltpu.sync_copy(ej.at[pl.ds(off, C)], idxj)
        cp1 = pltpu.async_copy(nf.at[idxi], rowsi, sem)
        cp2 = pltpu.async_copy(nf.at[idxj], rowsj, sem)
        cp1.wait()
        cp2.wait()

        def grp(g, carry2):
            rows = g * 16 + iota
            xi = plsc.load_gather(rowsi, [rows, c0])
            yi = plsc.load_gather(rowsi, [rows, c1])
            zi = plsc.load_gather(rowsi, [rows, c2])
            ti = plsc.load_gather(rowsi, [rows, c3])
            xj = plsc.load_gather(rowsj, [rows, c0])
            yj = plsc.load_gather(rowsj, [rows, c1])
            zj = plsc.load_gather(rowsj, [rows, c2])
            tj = plsc.load_gather(rowsj, [rows, c3])
            dx = xi - xj
            dy = yi - yj
            dz = zi - zj
            dt = ti - tj
            sl = pl.ds(g * 16, 16)
            bdt[sl] = dt
            bdx[sl] = dx
            bdy[sl] = dy
            bdz[sl] = dz
            return carry2

        lax.fori_loop(0, G, grp, 0)
        pltpu.sync_copy(bdt, feat.at[0, pl.ds(off, C)])
        pltpu.sync_copy(bdx, feat.at[1, pl.ds(off, C)])
        pltpu.sync_copy(bdy, feat.at[2, pl.ds(off, C)])
        pltpu.sync_copy(bdz, feat.at[3, pl.ds(off, C)])
        return carry

    lax.fori_loop(0, NWIN, win_body, 0)


_gather = functools.partial(
    pl.kernel,
    mesh=_MESH,
    out_type=jax.ShapeDtypeStruct((4, E_PAD), jnp.float32),
    scratch_types=[
        pltpu.VMEM((C,), jnp.int32),
        pltpu.VMEM((C,), jnp.int32),
        pltpu.VMEM((C, 8), jnp.float32),
        pltpu.VMEM((C, 8), jnp.float32),
        pltpu.VMEM((C,), jnp.float32),
        pltpu.VMEM((C,), jnp.float32),
        pltpu.VMEM((C,), jnp.float32),
        pltpu.VMEM((C,), jnp.float32),
        pltpu.SemaphoreType.DMA,
    ],
    compiler_params=_SC_PARAMS,
)(_gather_body)


def _mlp_body(feat_ref, w1_ref, b1_ref, w2_ref, b2_ref, w3_ref, b3_ref,
              out_ref):
    dn = (((1,), (0,)), ((), ()))
    prec = lax.Precision.DEFAULT
    dxv = feat_ref[1:2, :]
    dyv = feat_ref[2:3, :]
    dzv = feat_ref[3:4, :]
    r2 = dxv * dxv + dyv * dyv + dzv * dzv
    x = jnp.concatenate([feat_ref[0:1, :], r2], axis=0)
    h = lax.dot_general(w1_ref[...], x, dn, precision=prec,
                        preferred_element_type=jnp.float32)
    h = jnp.maximum(h + b1_ref[...][:, None], 0.0)
    h = lax.dot_general(w2_ref[...], h, dn, precision=prec,
                        preferred_element_type=jnp.float32)
    h = jnp.maximum(h + b2_ref[...][:, None], 0.0)
    wg = lax.dot_general(w3_ref[...], h, dn, precision=prec,
                         preferred_element_type=jnp.float32)
    wg = wg + b3_ref[...][:, None]
    rn = lax.rsqrt(jnp.maximum(r2, 1e-24))
    out_ref[...] = wg * rn


def _mlp(feat, w1, b1, w2, b2, w3, b3):
    return pl.pallas_call(
        _mlp_body,
        grid=(E_PAD // BC,),
        in_specs=[
            pl.BlockSpec((4, BC), lambda i: (0, i)),
            pl.BlockSpec((20, 2), lambda i: (0, 0)),
            pl.BlockSpec((20,), lambda i: (0,)),
            pl.BlockSpec((20, 20), lambda i: (0, 0)),
            pl.BlockSpec((20,), lambda i: (0,)),
            pl.BlockSpec((1, 20), lambda i: (0, 0)),
            pl.BlockSpec((1,), lambda i: (0,)),
        ],
        out_specs=pl.BlockSpec((1, BC), lambda i: (0, i)),
        out_shape=jax.ShapeDtypeStruct((1, E_PAD), jnp.float32),
    )(feat, w1, b1, w2, b2, w3, b3)


def _scatter_body(sv, feat, ei, parts, idxv, bs, bx, by, bz, aos, zb, acc):
    c = lax.axis_index("c")
    s = lax.axis_index("s")
    w = _worker_id()
    iota = lax.iota(jnp.int32, 16)
    zeros16 = jnp.zeros((16,), jnp.float32)
    c0 = jnp.zeros((16,), jnp.int32)
    c1 = c0 + 1
    c2 = c0 + 2
    c3 = c0 + 3

    def zfill(k, carry):
        o = k * 16 + iota
        plsc.store_scatter(zb, [lax.shift_right_logical(o, 3),
                                lax.bitwise_and(o, 7)], zeros16)
        return carry

    lax.fori_loop(0, RT * 8 // 16, zfill, 0)
    pltpu.sync_copy(zb, acc.at[pl.ds(s * RT, RT)])

    def z3(k, carry):
        o = k * 16 + iota
        plsc.store_scatter(aos, [lax.shift_right_logical(o, 3),
                                 lax.bitwise_and(o, 7)], zeros16)
        return carry

    lax.fori_loop(0, C * 8 // 16, z3, 0)
    plsc.subcore_barrier()

    base = w * EW

    def win_body(win, carry):
        off = base + win * C
        pltpu.sync_copy(ei.at[pl.ds(off, C)], idxv)
        pltpu.sync_copy(sv.at[0, pl.ds(off, C)], bs)
        pltpu.sync_copy(feat.at[1, pl.ds(off, C)], bx)
        pltpu.sync_copy(feat.at[2, pl.ds(off, C)], by)
        pltpu.sync_copy(feat.at[3, pl.ds(off, C)], bz)

        def grp(g, carry2):
            rows = g * 16 + iota
            sl = pl.ds(g * 16, 16)
            s16 = bs[sl]
            plsc.store_scatter(aos, [rows, c0], bx[sl] * s16)
            plsc.store_scatter(aos, [rows, c1], by[sl] * s16)
            plsc.store_scatter(aos, [rows, c2], bz[sl] * s16)
            return carry2

        lax.fori_loop(0, G, grp, 0)
        pltpu.sync_copy(aos, acc.at[idxv], add=True)
        return carry

    lax.fori_loop(0, NWIN, win_body, 0)
    plsc.subcore_barrier()
    pltpu.sync_copy(acc.at[pl.ds(s * RT, RT)],
                    parts.at[c, pl.ds(s * RT, RT), :])


_scatter = functools.partial(
    pl.kernel,
    mesh=_MESH,
    out_type=jax.ShapeDtypeStruct((NC, N_PAD, 8), jnp.float32),
    scratch_types=[
        pltpu.VMEM((C,), jnp.int32),
        pltpu.VMEM((C,), jnp.float32),
        pltpu.VMEM((C,), jnp.float32),
        pltpu.VMEM((C,), jnp.float32),
        pltpu.VMEM((C,), jnp.float32),
        pltpu.VMEM((C, 8), jnp.float32),
        pltpu.VMEM((RT, 8), jnp.float32),
        pltpu.VMEM_SHARED((N_PAD, 8), jnp.float32),
    ],
    compiler_params=_SC_PARAMS,
)(_scatter_body)


def _combine_body(parts, out, b0, b1, bo):
    w = _worker_id()
    r0 = w * RW
    iota = lax.iota(jnp.int32, 16)
    pltpu.sync_copy(parts.at[0, pl.ds(r0, RW), :], b0)
    pltpu.sync_copy(parts.at[1, pl.ds(r0, RW), :], b1)

    def grp(k, carry):
        o = k * 16 + iota
        rows = lax.div(o, 3)
        cols = o - rows * 3
        v0 = plsc.load_gather(b0, [rows, cols])
        v1 = plsc.load_gather(b1, [rows, cols])
        plsc.store_scatter(bo, [rows, cols], v0 + v1)
        return carry

    lax.fori_loop(0, RW * 3 // 16, grp, 0)

    @pl.when(w != NW - 1)
    def _():
        pltpu.sync_copy(bo, out.at[pl.ds(r0, RW), :])

    @pl.when(w == NW - 1)
    def _():
        pltpu.sync_copy(bo.at[pl.ds(0, R_LAST), :],
                        out.at[pl.ds((NW - 1) * RW, R_LAST), :])


_combine = functools.partial(
    pl.kernel,
    mesh=_MESH,
    out_type=jax.ShapeDtypeStruct((N, 3), jnp.float32),
    scratch_types=[
        pltpu.VMEM((RW, 8), jnp.float32),
        pltpu.VMEM((RW, 8), jnp.float32),
        pltpu.VMEM((RW, 3), jnp.float32),
    ],
    compiler_params=_SC_PARAMS,
)(_combine_body)


def kernel(t, pos, edge_index, W1, b1, W2, b2, W3, b3):
    nf = jnp.concatenate([pos, t, jnp.zeros((N, 4), jnp.float32)],
                         axis=1)
    pad = jnp.arange(E_PAD - E, dtype=edge_index.dtype) % N
    ei = jnp.concatenate([edge_index[0], pad])
    ej = jnp.concatenate([edge_index[1], pad])
    feat = _gather(nf, ei, ej)
    sv = _mlp(feat, W1, b1, W2, b2, W3, b3)
    parts = _scatter(sv, feat, ei)
    return _combine(parts)


# speedup vs baseline: 61.7561x; 2.0151x over previous
import functools

import jax
import jax.numpy as jnp
from jax import lax
from jax.experimental import pallas as pl
from jax.experimental.pallas import tpu as pltpu
from jax.experimental.pallas import tpu_sc as plsc

N = 50000
E = 1600000

NC = 2
NS = 16
NW = NC * NS

E_PAD = 1638400
EW = E_PAD // NW
C = 2048
NWIN = EW // C
G = C // 16

N_PAD = 50176
RT = N_PAD // NS
RW = N_PAD // NW
R_LAST = N - (NW - 1) * RW

BC = 32768

_MESH = plsc.VectorSubcoreMesh(core_axis_name="c", subcore_axis_name="s")
_SC_PARAMS = pltpu.CompilerParams(needs_layout_passes=False, use_tc_tiling_on_sc=False)


def _worker_id():
    return lax.axis_index("s") * NC + lax.axis_index("c")


def _gather_body(nf, ei, ej, odt, odx, ody, odz, idxi, idxj, rowsi, rowsj,
                 bdt, bdx, bdy, bdz, sem):
    w = _worker_id()
    base = w * EW
    iota = lax.iota(jnp.int32, 16)
    c0 = jnp.zeros((16,), jnp.int32)
    c1 = c0 + 1
    c2 = c0 + 2
    c3 = c0 + 3

    def win_body(win, carry):
        off = base + win * C
        pltpu.sync_copy(ei.at[pl.ds(off, C)], idxi)
        pltpu.sync_copy(ej.at[pl.ds(off, C)], idxj)
        cp1 = pltpu.async_copy(nf.at[idxi], rowsi, sem)
        cp2 = pltpu.async_copy(nf.at[idxj], rowsj, sem)
        cp1.wait()
        cp2.wait()

        def grp(g, carry2):
            rows = g * 16 + iota
            xi = plsc.load_gather(rowsi, [rows, c0])
            yi = plsc.load_gather(rowsi, [rows, c1])
            zi = plsc.load_gather(rowsi, [rows, c2])
            ti = plsc.load_gather(rowsi, [rows, c3])
            xj = plsc.load_gather(rowsj, [rows, c0])
            yj = plsc.load_gather(rowsj, [rows, c1])
            zj = plsc.load_gather(rowsj, [rows, c2])
            tj = plsc.load_gather(rowsj, [rows, c3])
            dx = xi - xj
            dy = yi - yj
            dz = zi - zj
            dt = ti - tj
            sl = pl.ds(g * 16, 16)
            bdt[sl] = dt
            bdx[sl] = dx
            bdy[sl] = dy
            bdz[sl] = dz
            return carry2

        lax.fori_loop(0, G, grp, 0)
        pltpu.sync_copy(bdt, odt.at[pl.ds(off, C)])
        pltpu.sync_copy(bdx, odx.at[pl.ds(off, C)])
        pltpu.sync_copy(bdy, ody.at[pl.ds(off, C)])
        pltpu.sync_copy(bdz, odz.at[pl.ds(off, C)])
        return carry

    lax.fori_loop(0, NWIN, win_body, 0)


_gather = functools.partial(
    pl.kernel,
    mesh=_MESH,
    out_type=[jax.ShapeDtypeStruct((E_PAD,), jnp.float32)] * 4,
    scratch_types=[
        pltpu.VMEM((C,), jnp.int32),
        pltpu.VMEM((C,), jnp.int32),
        pltpu.VMEM((C, 8), jnp.float32),
        pltpu.VMEM((C, 8), jnp.float32),
        pltpu.VMEM((C,), jnp.float32),
        pltpu.VMEM((C,), jnp.float32),
        pltpu.VMEM((C,), jnp.float32),
        pltpu.VMEM((C,), jnp.float32),
        pltpu.SemaphoreType.DMA,
    ],
    compiler_params=_SC_PARAMS,
)(_gather_body)


R128 = E_PAD // 128
BR = 256


def _mlp_body(dt_ref, dx_ref, dy_ref, dz_ref, w1_ref, b1_ref, w2_ref,
              b2_ref, w3_ref, b3_ref, out_ref):
    dn = (((1,), (0,)), ((), ()))
    prec = lax.Precision.DEFAULT
    n = BR * 128
    dxv = dx_ref[...]
    dyv = dy_ref[...]
    dzv = dz_ref[...]
    r2 = dxv * dxv + dyv * dyv + dzv * dzv
    dt1 = jnp.reshape(dt_ref[...], (1, n))
    r21 = jnp.reshape(r2, (1, n))
    x = jnp.concatenate([dt1, r21], axis=0)
    h = lax.dot_general(w1_ref[...], x, dn, precision=prec,
                        preferred_element_type=jnp.float32)
    h = jnp.maximum(h + b1_ref[...][:, None], 0.0)
    h = lax.dot_general(w2_ref[...], h, dn, precision=prec,
                        preferred_element_type=jnp.float32)
    h = jnp.maximum(h + b2_ref[...][:, None], 0.0)
    wg = lax.dot_general(w3_ref[...], h, dn, precision=prec,
                         preferred_element_type=jnp.float32)
    wg = wg + b3_ref[...][:, None]
    rn = lax.rsqrt(jnp.maximum(r21, 1e-24))
    out_ref[...] = jnp.reshape(wg * rn, (BR, 128))


def _mlp(dt, dx, dy, dz, w1, b1, w2, b2, w3, b3):
    return pl.pallas_call(
        _mlp_body,
        grid=(R128 // BR,),
        in_specs=[pl.BlockSpec((BR, 128), lambda i: (i, 0))] * 4 + [
            pl.BlockSpec((20, 2), lambda i: (0, 0)),
            pl.BlockSpec((20,), lambda i: (0,)),
            pl.BlockSpec((20, 20), lambda i: (0, 0)),
            pl.BlockSpec((20,), lambda i: (0,)),
            pl.BlockSpec((1, 20), lambda i: (0, 0)),
            pl.BlockSpec((1,), lambda i: (0,)),
        ],
        out_specs=pl.BlockSpec((BR, 128), lambda i: (i, 0)),
        out_shape=jax.ShapeDtypeStruct((R128, 128), jnp.float32),
    )(dt, dx, dy, dz, w1, b1, w2, b2, w3, b3)


def _scatter_body(sv, dxa, dya, dza, ei, parts, idxv, bs, bx, by, bz, aos, zb, acc):
    c = lax.axis_index("c")
    s = lax.axis_index("s")
    w = _worker_id()
    iota = lax.iota(jnp.int32, 16)
    zeros16 = jnp.zeros((16,), jnp.float32)
    c0 = jnp.zeros((16,), jnp.int32)
    c1 = c0 + 1
    c2 = c0 + 2
    c3 = c0 + 3

    def zfill(k, carry):
        o = k * 16 + iota
        plsc.store_scatter(zb, [lax.shift_right_logical(o, 3),
                                lax.bitwise_and(o, 7)], zeros16)
        return carry

    lax.fori_loop(0, RT * 8 // 16, zfill, 0)
    pltpu.sync_copy(zb, acc.at[pl.ds(s * RT, RT)])

    def z3(k, carry):
        o = k * 16 + iota
        plsc.store_scatter(aos, [lax.shift_right_logical(o, 3),
                                 lax.bitwise_and(o, 7)], zeros16)
        return carry

    lax.fori_loop(0, C * 8 // 16, z3, 0)
    plsc.subcore_barrier()

    base = w * EW

    def win_body(win, carry):
        off = base + win * C
        pltpu.sync_copy(ei.at[pl.ds(off, C)], idxv)
        pltpu.sync_copy(sv.at[pl.ds(off, C)], bs)
        pltpu.sync_copy(dxa.at[pl.ds(off, C)], bx)
        pltpu.sync_copy(dya.at[pl.ds(off, C)], by)
        pltpu.sync_copy(dza.at[pl.ds(off, C)], bz)

        def grp(g, carry2):
            rows = g * 16 + iota
            sl = pl.ds(g * 16, 16)
            s16 = bs[sl]
            plsc.store_scatter(aos, [rows, c0], bx[sl] * s16)
            plsc.store_scatter(aos, [rows, c1], by[sl] * s16)
            plsc.store_scatter(aos, [rows, c2], bz[sl] * s16)
            return carry2

        lax.fori_loop(0, G, grp, 0)
        pltpu.sync_copy(aos, acc.at[idxv], add=True)
        return carry

    lax.fori_loop(0, NWIN, win_body, 0)
    plsc.subcore_barrier()
    pltpu.sync_copy(acc.at[pl.ds(s * RT, RT)],
                    parts.at[c, pl.ds(s * RT, RT), :])


_scatter = functools.partial(
    pl.kernel,
    mesh=_MESH,
    out_type=jax.ShapeDtypeStruct((NC, N_PAD, 8), jnp.float32),
    scratch_types=[
        pltpu.VMEM((C,), jnp.int32),
        pltpu.VMEM((C,), jnp.float32),
        pltpu.VMEM((C,), jnp.float32),
        pltpu.VMEM((C,), jnp.float32),
        pltpu.VMEM((C,), jnp.float32),
        pltpu.VMEM((C, 8), jnp.float32),
        pltpu.VMEM((RT, 8), jnp.float32),
        pltpu.VMEM_SHARED((N_PAD, 8), jnp.float32),
    ],
    compiler_params=_SC_PARAMS,
)(_scatter_body)


def _combine_body(parts, out, b0, b1, bo):
    w = _worker_id()
    r0 = w * RW
    iota = lax.iota(jnp.int32, 16)
    pltpu.sync_copy(parts.at[0, pl.ds(r0, RW), :], b0)
    pltpu.sync_copy(parts.at[1, pl.ds(r0, RW), :], b1)

    def grp(k, carry):
        o = k * 16 + iota
        rows = lax.div(o, 3)
        cols = o - rows * 3
        v0 = plsc.load_gather(b0, [rows, cols])
        v1 = plsc.load_gather(b1, [rows, cols])
        plsc.store_scatter(bo, [rows, cols], v0 + v1)
        return carry

    lax.fori_loop(0, RW * 3 // 16, grp, 0)

    @pl.when(w != NW - 1)
    def _():
        pltpu.sync_copy(bo, out.at[pl.ds(r0, RW), :])

    @pl.when(w == NW - 1)
    def _():
        pltpu.sync_copy(bo.at[pl.ds(0, R_LAST), :],
                        out.at[pl.ds((NW - 1) * RW, R_LAST), :])


_combine = functools.partial(
    pl.kernel,
    mesh=_MESH,
    out_type=jax.ShapeDtypeStruct((N, 3), jnp.float32),
    scratch_types=[
        pltpu.VMEM((RW, 8), jnp.float32),
        pltpu.VMEM((RW, 8), jnp.float32),
        pltpu.VMEM((RW, 3), jnp.float32),
    ],
    compiler_params=_SC_PARAMS,
)(_combine_body)


def kernel(t, pos, edge_index, W1, b1, W2, b2, W3, b3):
    nf = jnp.concatenate([pos, t, jnp.zeros((N, 4), jnp.float32)],
                         axis=1)
    pad = jnp.arange(E_PAD - E, dtype=edge_index.dtype) % N
    ei = jnp.concatenate([edge_index[0], pad])
    ej = jnp.concatenate([edge_index[1], pad])
    dt, dx, dy, dz = _gather(nf, ei, ej)
    s2 = _mlp(dt.reshape(R128, 128), dx.reshape(R128, 128),
              dy.reshape(R128, 128), dz.reshape(R128, 128),
              W1, b1, W2, b2, W3, b3)
    parts = _scatter(s2.reshape(E_PAD), dx, dy, dz, ei)
    return _combine(parts)


# speedup vs baseline: 84.1050x; 1.3619x over previous
import functools

import jax
import jax.numpy as jnp
from jax import lax
from jax.experimental import pallas as pl
from jax.experimental.pallas import tpu as pltpu
from jax.experimental.pallas import tpu_sc as plsc

N = 50000
E = 1600000

NC = 2
NS = 16
NW = NC * NS

E_PAD = 1638400
EW = E_PAD // NW
C = 2048
NWIN = EW // C
G = C // 16

N_PAD = 50176
RT = N_PAD // NS
RW = N_PAD // NW
R_LAST = N - (NW - 1) * RW

BC = 32768

_MESH = plsc.VectorSubcoreMesh(core_axis_name="c", subcore_axis_name="s")
_SC_PARAMS = pltpu.CompilerParams(needs_layout_passes=False, use_tc_tiling_on_sc=False)


def _worker_id():
    return lax.axis_index("s") * NC + lax.axis_index("c")


def _gather_body(nf, ei, ej, odt, odx, ody, odz, idxi2, idxj2,
                 rowsi2, rowsj2, bdt, bdx, bdy, bdz, semx, semg):
    w = _worker_id()
    base = w * EW
    iota = lax.iota(jnp.int32, 16)
    c0 = jnp.zeros((16,), jnp.int32)
    c1 = c0 + 1
    c2 = c0 + 2
    c3 = c0 + 3

    def win_sl(win):
        return pl.ds(base + win * C, C)

    def fire_idx(win, q):
        pltpu.async_copy(ei.at[win_sl(win)], idxi2.at[q], semx.at[q])
        pltpu.async_copy(ej.at[win_sl(win)], idxj2.at[q], semx.at[q])

    def wait_idx(win, q):
        pltpu.make_async_copy(ei.at[win_sl(win)], idxi2.at[q],
                              semx.at[q]).wait()
        pltpu.make_async_copy(ej.at[win_sl(win)], idxj2.at[q],
                              semx.at[q]).wait()

    def fire_gather(q):
        pltpu.async_copy(nf.at[idxi2.at[q]], rowsi2.at[q], semg.at[q])
        pltpu.async_copy(nf.at[idxj2.at[q]], rowsj2.at[q], semg.at[q])

    def wait_gather(q):
        pltpu.make_async_copy(nf.at[idxi2.at[q]], rowsi2.at[q],
                              semg.at[q]).wait()
        pltpu.make_async_copy(nf.at[idxj2.at[q]], rowsj2.at[q],
                              semg.at[q]).wait()

    fire_idx(0, 0)
    wait_idx(0, 0)
    fire_gather(0)
    fire_idx(1, 1)

    def win_body(win, carry):
        prty = lax.bitwise_and(win, 1)
        q = 1 - prty

        @pl.when(win < NWIN - 1)
        def _():
            wait_idx(win + 1, q)
            fire_gather(q)

        wait_gather(prty)

        @pl.when(win < NWIN - 2)
        def _():
            fire_idx(win + 2, prty)

        ri = rowsi2.at[prty]
        rj = rowsj2.at[prty]

        def grp(g, carry2):
            rows = g * 16 + iota
            xi = plsc.load_gather(ri, [rows, c0])
            yi = plsc.load_gather(ri, [rows, c1])
            zi = plsc.load_gather(ri, [rows, c2])
            ti = plsc.load_gather(ri, [rows, c3])
            xj = plsc.load_gather(rj, [rows, c0])
            yj = plsc.load_gather(rj, [rows, c1])
            zj = plsc.load_gather(rj, [rows, c2])
            tj = plsc.load_gather(rj, [rows, c3])
            dx = xi - xj
            dy = yi - yj
            dz = zi - zj
            dt = ti - tj
            sl = pl.ds(g * 16, 16)
            bdt[sl] = dt
            bdx[sl] = dx
            bdy[sl] = dy
            bdz[sl] = dz
            return carry2

        lax.fori_loop(0, G, grp, 0)
        pltpu.sync_copy(bdt, odt.at[win_sl(win)])
        pltpu.sync_copy(bdx, odx.at[win_sl(win)])
        pltpu.sync_copy(bdy, ody.at[win_sl(win)])
        pltpu.sync_copy(bdz, odz.at[win_sl(win)])
        return carry

    lax.fori_loop(0, NWIN, win_body, 0)


_gather = functools.partial(
    pl.kernel,
    mesh=_MESH,
    out_type=[jax.ShapeDtypeStruct((E_PAD,), jnp.float32)] * 4,
    scratch_types=[
        pltpu.VMEM((2, C), jnp.int32),
        pltpu.VMEM((2, C), jnp.int32),
        pltpu.VMEM((2, C, 8), jnp.float32),
        pltpu.VMEM((2, C, 8), jnp.float32),
        pltpu.VMEM((C,), jnp.float32),
        pltpu.VMEM((C,), jnp.float32),
        pltpu.VMEM((C,), jnp.float32),
        pltpu.VMEM((C,), jnp.float32),
        pltpu.SemaphoreType.DMA((2,)),
        pltpu.SemaphoreType.DMA((2,)),
    ],
    compiler_params=_SC_PARAMS,
)(_gather_body)


R128 = E_PAD // 128
BR = 256


def _mlp_body(dt_ref, dx_ref, dy_ref, dz_ref, w1_ref, b1_ref, w2_ref,
              b2_ref, w3_ref, b3_ref, out_ref):
    dn = (((1,), (0,)), ((), ()))
    prec = lax.Precision.DEFAULT
    n = BR * 128
    dxv = dx_ref[...]
    dyv = dy_ref[...]
    dzv = dz_ref[...]
    r2 = dxv * dxv + dyv * dyv + dzv * dzv
    dt1 = jnp.reshape(dt_ref[...], (1, n))
    r21 = jnp.reshape(r2, (1, n))
    x = jnp.concatenate([dt1, r21], axis=0)
    h = lax.dot_general(w1_ref[...], x, dn, precision=prec,
                        preferred_element_type=jnp.float32)
    h = jnp.maximum(h + b1_ref[...][:, None], 0.0)
    h = lax.dot_general(w2_ref[...], h, dn, precision=prec,
                        preferred_element_type=jnp.float32)
    h = jnp.maximum(h + b2_ref[...][:, None], 0.0)
    wg = lax.dot_general(w3_ref[...], h, dn, precision=prec,
                         preferred_element_type=jnp.float32)
    wg = wg + b3_ref[...][:, None]
    rn = lax.rsqrt(jnp.maximum(r21, 1e-24))
    out_ref[...] = jnp.reshape(wg * rn, (BR, 128))


def _mlp(dt, dx, dy, dz, w1, b1, w2, b2, w3, b3):
    return pl.pallas_call(
        _mlp_body,
        grid=(R128 // BR,),
        in_specs=[pl.BlockSpec((BR, 128), lambda i: (i, 0))] * 4 + [
            pl.BlockSpec((20, 2), lambda i: (0, 0)),
            pl.BlockSpec((20,), lambda i: (0,)),
            pl.BlockSpec((20, 20), lambda i: (0, 0)),
            pl.BlockSpec((20,), lambda i: (0,)),
            pl.BlockSpec((1, 20), lambda i: (0, 0)),
            pl.BlockSpec((1,), lambda i: (0,)),
        ],
        out_specs=pl.BlockSpec((BR, 128), lambda i: (i, 0)),
        out_shape=jax.ShapeDtypeStruct((R128, 128), jnp.float32),
    )(dt, dx, dy, dz, w1, b1, w2, b2, w3, b3)


def _scatter_body(sv, dxa, dya, dza, ei, parts, idxv2, bs2, bx2, by2, bz2,
                  aos2, zb, acc, semi, sems):
    c = lax.axis_index("c")
    s = lax.axis_index("s")
    w = _worker_id()
    iota = lax.iota(jnp.int32, 16)
    zeros16 = jnp.zeros((16,), jnp.float32)
    c0 = jnp.zeros((16,), jnp.int32)
    c1 = c0 + 1
    c2 = c0 + 2

    def zfill(k, carry):
        o = k * 16 + iota
        plsc.store_scatter(zb, [lax.shift_right_logical(o, 3),
                                lax.bitwise_and(o, 7)], zeros16)
        return carry

    lax.fori_loop(0, RT * 8 // 16, zfill, 0)
    pltpu.sync_copy(zb, acc.at[pl.ds(s * RT, RT)])

    for pp in range(2):
        def z3(k, carry, _pp=pp):
            o = k * 16 + iota
            plsc.store_scatter(aos2.at[_pp],
                               [lax.shift_right_logical(o, 3),
                                lax.bitwise_and(o, 7)], zeros16)
            return carry

        lax.fori_loop(0, C * 8 // 16, z3, 0)
    plsc.subcore_barrier()

    base = w * EW

    def win_sl(win):
        return pl.ds(base + win * C, C)

    def fire_in(win, q):
        pltpu.async_copy(ei.at[win_sl(win)], idxv2.at[q], semi.at[q])
        pltpu.async_copy(sv.at[win_sl(win)], bs2.at[q], semi.at[q])
        pltpu.async_copy(dxa.at[win_sl(win)], bx2.at[q], semi.at[q])
        pltpu.async_copy(dya.at[win_sl(win)], by2.at[q], semi.at[q])
        pltpu.async_copy(dza.at[win_sl(win)], bz2.at[q], semi.at[q])

    def wait_in(win, q):
        pltpu.make_async_copy(ei.at[win_sl(win)], idxv2.at[q],
                              semi.at[q]).wait()
        pltpu.make_async_copy(sv.at[win_sl(win)], bs2.at[q],
                              semi.at[q]).wait()
        pltpu.make_async_copy(dxa.at[win_sl(win)], bx2.at[q],
                              semi.at[q]).wait()
        pltpu.make_async_copy(dya.at[win_sl(win)], by2.at[q],
                              semi.at[q]).wait()
        pltpu.make_async_copy(dza.at[win_sl(win)], bz2.at[q],
                              semi.at[q]).wait()

    def fire_scat(q):
        pltpu.async_copy(aos2.at[q], acc.at[idxv2.at[q]], sems.at[q],
                         add=True)

    def wait_scat(q):
        pltpu.make_async_copy(aos2.at[q], acc.at[idxv2.at[q]],
                              sems.at[q]).wait()

    fire_in(0, 0)

    def win_body(win, carry):
        prty = lax.bitwise_and(win, 1)
        q = 1 - prty
        wait_in(win, prty)

        @pl.when(jnp.logical_and(win >= 1, win < NWIN - 1))
        def _():
            wait_scat(q)

        @pl.when(win < NWIN - 1)
        def _():
            fire_in(win + 1, q)

        ap = aos2.at[prty]
        bs = bs2.at[prty]
        bx = bx2.at[prty]
        by = by2.at[prty]
        bz = bz2.at[prty]

        def grp(g, carry2):
            rows = g * 16 + iota
            sl = pl.ds(g * 16, 16)
            s16 = bs[sl]
            plsc.store_scatter(ap, [rows, c0], bx[sl] * s16)
            plsc.store_scatter(ap, [rows, c1], by[sl] * s16)
            plsc.store_scatter(ap, [rows, c2], bz[sl] * s16)
            return carry2

        lax.fori_loop(0, G, grp, 0)
        fire_scat(prty)
        return carry

    lax.fori_loop(0, NWIN, win_body, 0)
    wait_scat(0)
    wait_scat(1)
    plsc.subcore_barrier()
    pltpu.sync_copy(acc.at[pl.ds(s * RT, RT)],
                    parts.at[c, pl.ds(s * RT, RT), :])


_scatter = functools.partial(
    pl.kernel,
    mesh=_MESH,
    out_type=jax.ShapeDtypeStruct((NC, N_PAD, 8), jnp.float32),
    scratch_types=[
        pltpu.VMEM((2, C), jnp.int32),
        pltpu.VMEM((2, C), jnp.float32),
        pltpu.VMEM((2, C), jnp.float32),
        pltpu.VMEM((2, C), jnp.float32),
        pltpu.VMEM((2, C), jnp.float32),
        pltpu.VMEM((2, C, 8), jnp.float32),
        pltpu.VMEM((RT, 8), jnp.float32),
        pltpu.VMEM_SHARED((N_PAD, 8), jnp.float32),
        pltpu.SemaphoreType.DMA((2,)),
        pltpu.SemaphoreType.DMA((2,)),
    ],
    compiler_params=_SC_PARAMS,
)(_scatter_body)


def _combine_body(parts, out, b0, b1, bo):
    w = _worker_id()
    r0 = w * RW
    iota = lax.iota(jnp.int32, 16)
    pltpu.sync_copy(parts.at[0, pl.ds(r0, RW), :], b0)
    pltpu.sync_copy(parts.at[1, pl.ds(r0, RW), :], b1)

    def grp(k, carry):
        o = k * 16 + iota
        rows = lax.div(o, 3)
        cols = o - rows * 3
        v0 = plsc.load_gather(b0, [rows, cols])
        v1 = plsc.load_gather(b1, [rows, cols])
        plsc.store_scatter(bo, [rows, cols], v0 + v1)
        return carry

    lax.fori_loop(0, RW * 3 // 16, grp, 0)

    @pl.when(w != NW - 1)
    def _():
        pltpu.sync_copy(bo, out.at[pl.ds(r0, RW), :])

    @pl.when(w == NW - 1)
    def _():
        pltpu.sync_copy(bo.at[pl.ds(0, R_LAST), :],
                        out.at[pl.ds((NW - 1) * RW, R_LAST), :])


_combine = functools.partial(
    pl.kernel,
    mesh=_MESH,
    out_type=jax.ShapeDtypeStruct((N, 3), jnp.float32),
    scratch_types=[
        pltpu.VMEM((RW, 8), jnp.float32),
        pltpu.VMEM((RW, 8), jnp.float32),
        pltpu.VMEM((RW, 3), jnp.float32),
    ],
    compiler_params=_SC_PARAMS,
)(_combine_body)


def kernel(t, pos, edge_index, W1, b1, W2, b2, W3, b3):
    nf = jnp.concatenate([pos, t, jnp.zeros((N, 4), jnp.float32)],
                         axis=1)
    pad = jnp.arange(E_PAD - E, dtype=edge_index.dtype) % N
    ei = jnp.concatenate([edge_index[0], pad])
    ej = jnp.concatenate([edge_index[1], pad])
    dt, dx, dy, dz = _gather(nf, ei, ej)
    s2 = _mlp(dt.reshape(R128, 128), dx.reshape(R128, 128),
              dy.reshape(R128, 128), dz.reshape(R128, 128),
              W1, b1, W2, b2, W3, b3)
    parts = _scatter(s2.reshape(E_PAD), dx, dy, dz, ei)
    return _combine(parts)


# speedup vs baseline: 84.2916x; 1.0022x over previous
import functools

import jax
import jax.numpy as jnp
from jax import lax
from jax.experimental import pallas as pl
from jax.experimental.pallas import tpu as pltpu
from jax.experimental.pallas import tpu_sc as plsc

N = 50000
E = 1600000

NC = 2
NS = 16
NW = NC * NS

E_PAD = 1638400
EW = E_PAD // NW
C = 2048
NWIN = EW // C
G = C // 16

N_PAD = 50176
RT = N_PAD // NS
RW = N_PAD // NW
R_LAST = N - (NW - 1) * RW

BC = 32768

_MESH = plsc.VectorSubcoreMesh(core_axis_name="c", subcore_axis_name="s")
_SC_PARAMS = pltpu.CompilerParams(needs_layout_passes=False, use_tc_tiling_on_sc=False)


def _worker_id():
    return lax.axis_index("s") * NC + lax.axis_index("c")


def _gather_body(nf, ei, ej, odt, odx, ody, odz, idxi2, idxj2,
                 rowsi2, rowsj2, bdt2, bdx2, bdy2, bdz2, semx, semg, semo):
    w = _worker_id()
    base = w * EW
    iota = lax.iota(jnp.int32, 16)
    c0 = jnp.zeros((16,), jnp.int32)
    c1 = c0 + 1
    c2 = c0 + 2
    c3 = c0 + 3

    def win_sl(win):
        return pl.ds(base + win * C, C)

    def fire_idx(win, q):
        pltpu.async_copy(ei.at[win_sl(win)], idxi2.at[q], semx.at[q])
        pltpu.async_copy(ej.at[win_sl(win)], idxj2.at[q], semx.at[q])

    def wait_idx(win, q):
        pltpu.make_async_copy(ei.at[win_sl(win)], idxi2.at[q],
                              semx.at[q]).wait()
        pltpu.make_async_copy(ej.at[win_sl(win)], idxj2.at[q],
                              semx.at[q]).wait()

    def fire_gather(q):
        pltpu.async_copy(nf.at[idxi2.at[q]], rowsi2.at[q], semg.at[q])
        pltpu.async_copy(nf.at[idxj2.at[q]], rowsj2.at[q], semg.at[q])

    def wait_gather(q):
        pltpu.make_async_copy(nf.at[idxi2.at[q]], rowsi2.at[q],
                              semg.at[q]).wait()
        pltpu.make_async_copy(nf.at[idxj2.at[q]], rowsj2.at[q],
                              semg.at[q]).wait()

    def fire_out(win, q):
        pltpu.async_copy(bdt2.at[q], odt.at[win_sl(win)], semo.at[q])
        pltpu.async_copy(bdx2.at[q], odx.at[win_sl(win)], semo.at[q])
        pltpu.async_copy(bdy2.at[q], ody.at[win_sl(win)], semo.at[q])
        pltpu.async_copy(bdz2.at[q], odz.at[win_sl(win)], semo.at[q])

    def wait_out(win, q):
        pltpu.make_async_copy(bdt2.at[q], odt.at[win_sl(win)],
                              semo.at[q]).wait()
        pltpu.make_async_copy(bdx2.at[q], odx.at[win_sl(win)],
                              semo.at[q]).wait()
        pltpu.make_async_copy(bdy2.at[q], ody.at[win_sl(win)],
                              semo.at[q]).wait()
        pltpu.make_async_copy(bdz2.at[q], odz.at[win_sl(win)],
                              semo.at[q]).wait()

    fire_idx(0, 0)
    wait_idx(0, 0)
    fire_gather(0)
    fire_idx(1, 1)

    def win_body(win, carry):
        prty = lax.bitwise_and(win, 1)
        q = 1 - prty

        @pl.when(win < NWIN - 1)
        def _():
            wait_idx(win + 1, q)
            fire_gather(q)

        wait_gather(prty)

        @pl.when(win < NWIN - 2)
        def _():
            fire_idx(win + 2, prty)

        @pl.when(win >= 2)
        def _():
            wait_out(win - 2, prty)

        ri = rowsi2.at[prty]
        rj = rowsj2.at[prty]
        bdt = bdt2.at[prty]
        bdx = bdx2.at[prty]
        bdy = bdy2.at[prty]
        bdz = bdz2.at[prty]

        def grp(g, carry2):
            rows = g * 16 + iota
            xi = plsc.load_gather(ri, [rows, c0])
            yi = plsc.load_gather(ri, [rows, c1])
            zi = plsc.load_gather(ri, [rows, c2])
            ti = plsc.load_gather(ri, [rows, c3])
            xj = plsc.load_gather(rj, [rows, c0])
            yj = plsc.load_gather(rj, [rows, c1])
            zj = plsc.load_gather(rj, [rows, c2])
            tj = plsc.load_gather(rj, [rows, c3])
            dx = xi - xj
            dy = yi - yj
            dz = zi - zj
            dt = ti - tj
            sl = pl.ds(g * 16, 16)
            bdt[sl] = dt
            bdx[sl] = dx
            bdy[sl] = dy
            bdz[sl] = dz
            return carry2

        lax.fori_loop(0, G, grp, 0)
        fire_out(win, prty)
        return carry

    lax.fori_loop(0, NWIN, win_body, 0)
    wait_out(NWIN - 2, (NWIN - 2) & 1)
    wait_out(NWIN - 1, (NWIN - 1) & 1)


_gather = functools.partial(
    pl.kernel,
    mesh=_MESH,
    out_type=[jax.ShapeDtypeStruct((E_PAD,), jnp.float32)] * 4,
    scratch_types=[
        pltpu.VMEM((2, C), jnp.int32),
        pltpu.VMEM((2, C), jnp.int32),
        pltpu.VMEM((2, C, 8), jnp.float32),
        pltpu.VMEM((2, C, 8), jnp.float32),
        pltpu.VMEM((2, C), jnp.float32),
        pltpu.VMEM((2, C), jnp.float32),
        pltpu.VMEM((2, C), jnp.float32),
        pltpu.VMEM((2, C), jnp.float32),
        pltpu.SemaphoreType.DMA((2,)),
        pltpu.SemaphoreType.DMA((2,)),
        pltpu.SemaphoreType.DMA((2,)),
    ],
    compiler_params=_SC_PARAMS,
)(_gather_body)


R128 = E_PAD // 128
BR = 256


def _mlp_body(dt_ref, dx_ref, dy_ref, dz_ref, w1_ref, b1_ref, w2_ref,
              b2_ref, w3_ref, b3_ref, out_ref):
    dn = (((1,), (0,)), ((), ()))
    prec = lax.Precision.DEFAULT
    n = BR * 128
    dxv = dx_ref[...]
    dyv = dy_ref[...]
    dzv = dz_ref[...]
    r2 = dxv * dxv + dyv * dyv + dzv * dzv
    dt1 = jnp.reshape(dt_ref[...], (1, n))
    r21 = jnp.reshape(r2, (1, n))
    x = jnp.concatenate([dt1, r21], axis=0)
    h = lax.dot_general(w1_ref[...], x, dn, precision=prec,
                        preferred_element_type=jnp.float32)
    h = jnp.maximum(h + b1_ref[...][:, None], 0.0)
    h = lax.dot_general(w2_ref[...], h, dn, precision=prec,
                        preferred_element_type=jnp.float32)
    h = jnp.maximum(h + b2_ref[...][:, None], 0.0)
    wg = lax.dot_general(w3_ref[...], h, dn, precision=prec,
                         preferred_element_type=jnp.float32)
    wg = wg + b3_ref[...][:, None]
    rn = lax.rsqrt(jnp.maximum(r21, 1e-24))
    out_ref[...] = jnp.reshape(wg * rn, (BR, 128))


def _mlp(dt, dx, dy, dz, w1, b1, w2, b2, w3, b3):
    return pl.pallas_call(
        _mlp_body,
        grid=(R128 // BR,),
        in_specs=[pl.BlockSpec((BR, 128), lambda i: (i, 0))] * 4 + [
            pl.BlockSpec((20, 2), lambda i: (0, 0)),
            pl.BlockSpec((20,), lambda i: (0,)),
            pl.BlockSpec((20, 20), lambda i: (0, 0)),
            pl.BlockSpec((20,), lambda i: (0,)),
            pl.BlockSpec((1, 20), lambda i: (0, 0)),
            pl.BlockSpec((1,), lambda i: (0,)),
        ],
        out_specs=pl.BlockSpec((BR, 128), lambda i: (i, 0)),
        out_shape=jax.ShapeDtypeStruct((R128, 128), jnp.float32),
    )(dt, dx, dy, dz, w1, b1, w2, b2, w3, b3)


def _scatter_body(sv, dxa, dya, dza, ei, parts, idxv2, bs2, bx2, by2, bz2,
                  aos2, zb, acc, semi, sems):
    c = lax.axis_index("c")
    s = lax.axis_index("s")
    w = _worker_id()
    iota = lax.iota(jnp.int32, 16)
    zeros16 = jnp.zeros((16,), jnp.float32)
    c0 = jnp.zeros((16,), jnp.int32)
    c1 = c0 + 1
    c2 = c0 + 2

    def zfill(k, carry):
        o = k * 16 + iota
        plsc.store_scatter(zb, [lax.shift_right_logical(o, 3),
                                lax.bitwise_and(o, 7)], zeros16)
        return carry

    lax.fori_loop(0, RT * 8 // 16, zfill, 0)
    pltpu.sync_copy(zb, acc.at[pl.ds(s * RT, RT)])

    for pp in range(2):
        def z3(k, carry, _pp=pp):
            o = k * 16 + iota
            plsc.store_scatter(aos2.at[_pp],
                               [lax.shift_right_logical(o, 3),
                                lax.bitwise_and(o, 7)], zeros16)
            return carry

        lax.fori_loop(0, C * 8 // 16, z3, 0)
    plsc.subcore_barrier()

    base = w * EW

    def win_sl(win):
        return pl.ds(base + win * C, C)

    def fire_in(win, q):
        pltpu.async_copy(ei.at[win_sl(win)], idxv2.at[q], semi.at[q])
        pltpu.async_copy(sv.at[win_sl(win)], bs2.at[q], semi.at[q])
        pltpu.async_copy(dxa.at[win_sl(win)], bx2.at[q], semi.at[q])
        pltpu.async_copy(dya.at[win_sl(win)], by2.at[q], semi.at[q])
        pltpu.async_copy(dza.at[win_sl(win)], bz2.at[q], semi.at[q])

    def wait_in(win, q):
        pltpu.make_async_copy(ei.at[win_sl(win)], idxv2.at[q],
                              semi.at[q]).wait()
        pltpu.make_async_copy(sv.at[win_sl(win)], bs2.at[q],
                              semi.at[q]).wait()
        pltpu.make_async_copy(dxa.at[win_sl(win)], bx2.at[q],
                              semi.at[q]).wait()
        pltpu.make_async_copy(dya.at[win_sl(win)], by2.at[q],
                              semi.at[q]).wait()
        pltpu.make_async_copy(dza.at[win_sl(win)], bz2.at[q],
                              semi.at[q]).wait()

    def fire_scat(q):
        pltpu.async_copy(aos2.at[q], acc.at[idxv2.at[q]], sems.at[q],
                         add=True)

    def wait_scat(q):
        pltpu.make_async_copy(aos2.at[q], acc.at[idxv2.at[q]],
                              sems.at[q]).wait()

    fire_in(0, 0)

    def win_body(win, carry):
        prty = lax.bitwise_and(win, 1)
        q = 1 - prty
        wait_in(win, prty)

        @pl.when(jnp.logical_and(win >= 1, win < NWIN - 1))
        def _():
            wait_scat(q)

        @pl.when(win < NWIN - 1)
        def _():
            fire_in(win + 1, q)

        ap = aos2.at[prty]
        bs = bs2.at[prty]
        bx = bx2.at[prty]
        by = by2.at[prty]
        bz = bz2.at[prty]

        def grp(g, carry2):
            rows = g * 16 + iota
            sl = pl.ds(g * 16, 16)
            s16 = bs[sl]
            plsc.store_scatter(ap, [rows, c0], bx[sl] * s16)
            plsc.store_scatter(ap, [rows, c1], by[sl] * s16)
            plsc.store_scatter(ap, [rows, c2], bz[sl] * s16)
            return carry2

        lax.fori_loop(0, G, grp, 0)
        fire_scat(prty)
        return carry

    lax.fori_loop(0, NWIN, win_body, 0)
    wait_scat(0)
    wait_scat(1)
    plsc.subcore_barrier()
    pltpu.sync_copy(acc.at[pl.ds(s * RT, RT)],
                    parts.at[c, pl.ds(s * RT, RT), :])


_scatter = functools.partial(
    pl.kernel,
    mesh=_MESH,
    out_type=jax.ShapeDtypeStruct((NC, N_PAD, 8), jnp.float32),
    scratch_types=[
        pltpu.VMEM((2, C), jnp.int32),
        pltpu.VMEM((2, C), jnp.float32),
        pltpu.VMEM((2, C), jnp.float32),
        pltpu.VMEM((2, C), jnp.float32),
        pltpu.VMEM((2, C), jnp.float32),
        pltpu.VMEM((2, C, 8), jnp.float32),
        pltpu.VMEM((RT, 8), jnp.float32),
        pltpu.VMEM_SHARED((N_PAD, 8), jnp.float32),
        pltpu.SemaphoreType.DMA((2,)),
        pltpu.SemaphoreType.DMA((2,)),
    ],
    compiler_params=_SC_PARAMS,
)(_scatter_body)


def _combine_body(parts, out, b0, b1, bo):
    w = _worker_id()
    r0 = w * RW
    iota = lax.iota(jnp.int32, 16)
    pltpu.sync_copy(parts.at[0, pl.ds(r0, RW), :], b0)
    pltpu.sync_copy(parts.at[1, pl.ds(r0, RW), :], b1)

    def grp(k, carry):
        o = k * 16 + iota
        rows = lax.div(o, 3)
        cols = o - rows * 3
        v0 = plsc.load_gather(b0, [rows, cols])
        v1 = plsc.load_gather(b1, [rows, cols])
        plsc.store_scatter(bo, [rows, cols], v0 + v1)
        return carry

    lax.fori_loop(0, RW * 3 // 16, grp, 0)

    @pl.when(w != NW - 1)
    def _():
        pltpu.sync_copy(bo, out.at[pl.ds(r0, RW), :])

    @pl.when(w == NW - 1)
    def _():
        pltpu.sync_copy(bo.at[pl.ds(0, R_LAST), :],
                        out.at[pl.ds((NW - 1) * RW, R_LAST), :])


_combine = functools.partial(
    pl.kernel,
    mesh=_MESH,
    out_type=jax.ShapeDtypeStruct((N, 3), jnp.float32),
    scratch_types=[
        pltpu.VMEM((RW, 8), jnp.float32),
        pltpu.VMEM((RW, 8), jnp.float32),
        pltpu.VMEM((RW, 3), jnp.float32),
    ],
    compiler_params=_SC_PARAMS,
)(_combine_body)


def kernel(t, pos, edge_index, W1, b1, W2, b2, W3, b3):
    nf = jnp.concatenate([pos, t, jnp.zeros((N, 4), jnp.float32)],
                         axis=1)
    pad = jnp.arange(E_PAD - E, dtype=edge_index.dtype) % N
    ei = jnp.concatenate([edge_index[0], pad])
    ej = jnp.concatenate([edge_index[1], pad])
    dt, dx, dy, dz = _gather(nf, ei, ej)
    s2 = _mlp(dt.reshape(R128, 128), dx.reshape(R128, 128),
              dy.reshape(R128, 128), dz.reshape(R128, 128),
              W1, b1, W2, b2, W3, b3)
    parts = _scatter(s2.reshape(E_PAD), dx, dy, dz, ei)
    return _combine(parts)


# speedup vs baseline: 106.6069x; 1.2647x over previous
import functools

import jax
import jax.numpy as jnp
from jax import lax
from jax.experimental import pallas as pl
from jax.experimental.pallas import tpu as pltpu
from jax.experimental.pallas import tpu_sc as plsc

N = 50000
E = 1600000

NC = 2
NS = 16
NW = NC * NS

E_PAD = 1638400
E_H = E_PAD // 2
EW = E_H // NW
C = 1280
NWIN = EW // C
G = C // 16

N_PAD = 50176
RT = N_PAD // NS
RW = N_PAD // NW
R_LAST = N - (NW - 1) * RW

BC = 32768

_MESH = plsc.VectorSubcoreMesh(core_axis_name="c", subcore_axis_name="s")
_SC_PARAMS = pltpu.CompilerParams(needs_layout_passes=False, use_tc_tiling_on_sc=False)


def _worker_id():
    return lax.axis_index("s") * NC + lax.axis_index("c")


def _gather_body(nf, ei, ej, odt, odx, ody, odz, idxi2, idxj2,
                 rowsi2, rowsj2, bdt2, bdx2, bdy2, bdz2, semx, semg, semo):
    w = _worker_id()
    base = w * EW
    iota = lax.iota(jnp.int32, 16)
    c0 = jnp.zeros((16,), jnp.int32)
    c1 = c0 + 1
    c2 = c0 + 2
    c3 = c0 + 3

    def win_sl(win):
        return pl.ds(base + win * C, C)

    def fire_idx(win, q):
        pltpu.async_copy(ei.at[win_sl(win)], idxi2.at[q], semx.at[q])
        pltpu.async_copy(ej.at[win_sl(win)], idxj2.at[q], semx.at[q])

    def wait_idx(win, q):
        pltpu.make_async_copy(ei.at[win_sl(win)], idxi2.at[q],
                              semx.at[q]).wait()
        pltpu.make_async_copy(ej.at[win_sl(win)], idxj2.at[q],
                              semx.at[q]).wait()

    def fire_gather(q):
        pltpu.async_copy(nf.at[idxi2.at[q]], rowsi2.at[q], semg.at[q])
        pltpu.async_copy(nf.at[idxj2.at[q]], rowsj2.at[q], semg.at[q])

    def wait_gather(q):
        pltpu.make_async_copy(nf.at[idxi2.at[q]], rowsi2.at[q],
                              semg.at[q]).wait()
        pltpu.make_async_copy(nf.at[idxj2.at[q]], rowsj2.at[q],
                              semg.at[q]).wait()

    def fire_out(win, q):
        pltpu.async_copy(bdt2.at[q], odt.at[win_sl(win)], semo.at[q])
        pltpu.async_copy(bdx2.at[q], odx.at[win_sl(win)], semo.at[q])
        pltpu.async_copy(bdy2.at[q], ody.at[win_sl(win)], semo.at[q])
        pltpu.async_copy(bdz2.at[q], odz.at[win_sl(win)], semo.at[q])

    def wait_out(win, q):
        pltpu.make_async_copy(bdt2.at[q], odt.at[win_sl(win)],
                              semo.at[q]).wait()
        pltpu.make_async_copy(bdx2.at[q], odx.at[win_sl(win)],
                              semo.at[q]).wait()
        pltpu.make_async_copy(bdy2.at[q], ody.at[win_sl(win)],
                              semo.at[q]).wait()
        pltpu.make_async_copy(bdz2.at[q], odz.at[win_sl(win)],
                              semo.at[q]).wait()

    fire_idx(0, 0)
    wait_idx(0, 0)
    fire_gather(0)
    fire_idx(1, 1)

    def win_body(win, carry):
        prty = lax.bitwise_and(win, 1)
        q = 1 - prty

        @pl.when(win < NWIN - 1)
        def _():
            wait_idx(win + 1, q)
            fire_gather(q)

        wait_gather(prty)

        @pl.when(win < NWIN - 2)
        def _():
            fire_idx(win + 2, prty)

        @pl.when(win >= 2)
        def _():
            wait_out(win - 2, prty)

        ri = rowsi2.at[prty]
        rj = rowsj2.at[prty]
        bdt = bdt2.at[prty]
        bdx = bdx2.at[prty]
        bdy = bdy2.at[prty]
        bdz = bdz2.at[prty]

        def grp(g, carry2):
            rows = g * 16 + iota
            xi = plsc.load_gather(ri, [rows, c0])
            yi = plsc.load_gather(ri, [rows, c1])
            zi = plsc.load_gather(ri, [rows, c2])
            ti = plsc.load_gather(ri, [rows, c3])
            xj = plsc.load_gather(rj, [rows, c0])
            yj = plsc.load_gather(rj, [rows, c1])
            zj = plsc.load_gather(rj, [rows, c2])
            tj = plsc.load_gather(rj, [rows, c3])
            dx = xi - xj
            dy = yi - yj
            dz = zi - zj
            dt = ti - tj
            sl = pl.ds(g * 16, 16)
            bdt[sl] = dt
            bdx[sl] = dx
            bdy[sl] = dy
            bdz[sl] = dz
            return carry2

        lax.fori_loop(0, G, grp, 0)
        fire_out(win, prty)
        return carry

    lax.fori_loop(0, NWIN, win_body, 0)
    wait_out(NWIN - 2, (NWIN - 2) & 1)
    wait_out(NWIN - 1, (NWIN - 1) & 1)


_gather = functools.partial(
    pl.kernel,
    mesh=_MESH,
    out_type=[jax.ShapeDtypeStruct((E_H,), jnp.float32)] * 4,
    scratch_types=[
        pltpu.VMEM((2, C), jnp.int32),
        pltpu.VMEM((2, C), jnp.int32),
        pltpu.VMEM((2, C, 8), jnp.float32),
        pltpu.VMEM((2, C, 8), jnp.float32),
        pltpu.VMEM((2, C), jnp.float32),
        pltpu.VMEM((2, C), jnp.float32),
        pltpu.VMEM((2, C), jnp.float32),
        pltpu.VMEM((2, C), jnp.float32),
        pltpu.SemaphoreType.DMA((2,)),
        pltpu.SemaphoreType.DMA((2,)),
        pltpu.SemaphoreType.DMA((2,)),
    ],
    compiler_params=_SC_PARAMS,
)(_gather_body)


R128 = E_H // 128
BR = 256


def _mlp_body(dt_ref, dx_ref, dy_ref, dz_ref, w1_ref, b1_ref, w2_ref,
              b2_ref, w3_ref, b3_ref, out_ref):
    dn = (((1,), (0,)), ((), ()))
    prec = lax.Precision.DEFAULT
    n = BR * 128
    dxv = dx_ref[...]
    dyv = dy_ref[...]
    dzv = dz_ref[...]
    r2 = dxv * dxv + dyv * dyv + dzv * dzv
    dt1 = jnp.reshape(dt_ref[...], (1, n))
    r21 = jnp.reshape(r2, (1, n))
    x = jnp.concatenate([dt1, r21], axis=0)
    h = lax.dot_general(w1_ref[...], x, dn, precision=prec,
                        preferred_element_type=jnp.float32)
    h = jnp.maximum(h + b1_ref[...][:, None], 0.0)
    h = lax.dot_general(w2_ref[...], h, dn, precision=prec,
                        preferred_element_type=jnp.float32)
    h = jnp.maximum(h + b2_ref[...][:, None], 0.0)
    wg = lax.dot_general(w3_ref[...], h, dn, precision=prec,
                         preferred_element_type=jnp.float32)
    wg = wg + b3_ref[...][:, None]
    rn = lax.rsqrt(jnp.maximum(r21, 1e-24))
    out_ref[...] = jnp.reshape(wg * rn, (BR, 128))


def _mlp(dt, dx, dy, dz, w1, b1, w2, b2, w3, b3):
    return pl.pallas_call(
        _mlp_body,
        grid=(R128 // BR,),
        in_specs=[pl.BlockSpec((BR, 128), lambda i: (i, 0))] * 4 + [
            pl.BlockSpec((20, 2), lambda i: (0, 0)),
            pl.BlockSpec((20,), lambda i: (0,)),
            pl.BlockSpec((20, 20), lambda i: (0, 0)),
            pl.BlockSpec((20,), lambda i: (0,)),
            pl.BlockSpec((1, 20), lambda i: (0, 0)),
            pl.BlockSpec((1,), lambda i: (0,)),
        ],
        out_specs=pl.BlockSpec((BR, 128), lambda i: (i, 0)),
        out_shape=jax.ShapeDtypeStruct((R128, 128), jnp.float32),
    )(dt, dx, dy, dz, w1, b1, w2, b2, w3, b3)


def _scatter_body(sv, dxa, dya, dza, ei, parts, idxv2, bs2, bx2, by2, bz2,
                  aos2, zb, acc, semi, sems):
    c = lax.axis_index("c")
    s = lax.axis_index("s")
    w = _worker_id()
    iota = lax.iota(jnp.int32, 16)
    zeros16 = jnp.zeros((16,), jnp.float32)
    c0 = jnp.zeros((16,), jnp.int32)
    c1 = c0 + 1
    c2 = c0 + 2

    def zfill(k, carry):
        o = k * 16 + iota
        plsc.store_scatter(zb, [lax.shift_right_logical(o, 3),
                                lax.bitwise_and(o, 7)], zeros16)
        return carry

    lax.fori_loop(0, RT * 8 // 16, zfill, 0)
    pltpu.sync_copy(zb, acc.at[pl.ds(s * RT, RT)])

    for pp in range(2):
        def z3(k, carry, _pp=pp):
            o = k * 16 + iota
            plsc.store_scatter(aos2.at[_pp],
                               [lax.shift_right_logical(o, 3),
                                lax.bitwise_and(o, 7)], zeros16)
            return carry

        lax.fori_loop(0, C * 8 // 16, z3, 0)
    plsc.subcore_barrier()

    base = w * EW

    def win_sl(win):
        return pl.ds(base + win * C, C)

    def fire_in(win, q):
        pltpu.async_copy(ei.at[win_sl(win)], idxv2.at[q], semi.at[q])
        pltpu.async_copy(sv.at[win_sl(win)], bs2.at[q], semi.at[q])
        pltpu.async_copy(dxa.at[win_sl(win)], bx2.at[q], semi.at[q])
        pltpu.async_copy(dya.at[win_sl(win)], by2.at[q], semi.at[q])
        pltpu.async_copy(dza.at[win_sl(win)], bz2.at[q], semi.at[q])

    def wait_in(win, q):
        pltpu.make_async_copy(ei.at[win_sl(win)], idxv2.at[q],
                              semi.at[q]).wait()
        pltpu.make_async_copy(sv.at[win_sl(win)], bs2.at[q],
                              semi.at[q]).wait()
        pltpu.make_async_copy(dxa.at[win_sl(win)], bx2.at[q],
                              semi.at[q]).wait()
        pltpu.make_async_copy(dya.at[win_sl(win)], by2.at[q],
                              semi.at[q]).wait()
        pltpu.make_async_copy(dza.at[win_sl(win)], bz2.at[q],
                              semi.at[q]).wait()

    def fire_scat(q):
        pltpu.async_copy(aos2.at[q], acc.at[idxv2.at[q]], sems.at[q],
                         add=True)

    def wait_scat(q):
        pltpu.make_async_copy(aos2.at[q], acc.at[idxv2.at[q]],
                              sems.at[q]).wait()

    fire_in(0, 0)

    def win_body(win, carry):
        prty = lax.bitwise_and(win, 1)
        q = 1 - prty
        wait_in(win, prty)

        @pl.when(jnp.logical_and(win >= 1, win < NWIN - 1))
        def _():
            wait_scat(q)

        @pl.when(win < NWIN - 1)
        def _():
            fire_in(win + 1, q)

        ap = aos2.at[prty]
        bs = bs2.at[prty]
        bx = bx2.at[prty]
        by = by2.at[prty]
        bz = bz2.at[prty]

        def grp(g, carry2):
            rows = g * 16 + iota
            sl = pl.ds(g * 16, 16)
            s16 = bs[sl]
            plsc.store_scatter(ap, [rows, c0], bx[sl] * s16)
            plsc.store_scatter(ap, [rows, c1], by[sl] * s16)
            plsc.store_scatter(ap, [rows, c2], bz[sl] * s16)
            return carry2

        lax.fori_loop(0, G, grp, 0)
        fire_scat(prty)
        return carry

    lax.fori_loop(0, NWIN, win_body, 0)
    wait_scat(0)
    wait_scat(1)
    plsc.subcore_barrier()
    pltpu.sync_copy(acc.at[pl.ds(s * RT, RT)],
                    parts.at[c, pl.ds(s * RT, RT), :])


_scatter = functools.partial(
    pl.kernel,
    mesh=_MESH,
    out_type=jax.ShapeDtypeStruct((NC, N_PAD, 8), jnp.float32),
    scratch_types=[
        pltpu.VMEM((2, C), jnp.int32),
        pltpu.VMEM((2, C), jnp.float32),
        pltpu.VMEM((2, C), jnp.float32),
        pltpu.VMEM((2, C), jnp.float32),
        pltpu.VMEM((2, C), jnp.float32),
        pltpu.VMEM((2, C, 8), jnp.float32),
        pltpu.VMEM((RT, 8), jnp.float32),
        pltpu.VMEM_SHARED((N_PAD, 8), jnp.float32),
        pltpu.SemaphoreType.DMA((2,)),
        pltpu.SemaphoreType.DMA((2,)),
    ],
    compiler_params=_SC_PARAMS,
)(_scatter_body)


def _combine_body(pa, pb, out, b0, b1, b2, b3, bo):
    w = _worker_id()
    r0 = w * RW
    iota = lax.iota(jnp.int32, 16)
    pltpu.sync_copy(pa.at[0, pl.ds(r0, RW), :], b0)
    pltpu.sync_copy(pa.at[1, pl.ds(r0, RW), :], b1)
    pltpu.sync_copy(pb.at[0, pl.ds(r0, RW), :], b2)
    pltpu.sync_copy(pb.at[1, pl.ds(r0, RW), :], b3)

    def grp(k, carry):
        o = k * 16 + iota
        rows = lax.div(o, 3)
        cols = o - rows * 3
        v0 = plsc.load_gather(b0, [rows, cols])
        v1 = plsc.load_gather(b1, [rows, cols])
        v2 = plsc.load_gather(b2, [rows, cols])
        v3 = plsc.load_gather(b3, [rows, cols])
        plsc.store_scatter(bo, [rows, cols], (v0 + v1) + (v2 + v3))
        return carry

    lax.fori_loop(0, RW * 3 // 16, grp, 0)

    @pl.when(w != NW - 1)
    def _():
        pltpu.sync_copy(bo, out.at[pl.ds(r0, RW), :])

    @pl.when(w == NW - 1)
    def _():
        pltpu.sync_copy(bo.at[pl.ds(0, R_LAST), :],
                        out.at[pl.ds((NW - 1) * RW, R_LAST), :])


_combine = functools.partial(
    pl.kernel,
    mesh=_MESH,
    out_type=jax.ShapeDtypeStruct((N, 3), jnp.float32),
    scratch_types=[
        pltpu.VMEM((RW, 8), jnp.float32),
        pltpu.VMEM((RW, 8), jnp.float32),
        pltpu.VMEM((RW, 8), jnp.float32),
        pltpu.VMEM((RW, 8), jnp.float32),
        pltpu.VMEM((RW, 3), jnp.float32),
    ],
    compiler_params=_SC_PARAMS,
)(_combine_body)


def kernel(t, pos, edge_index, W1, b1, W2, b2, W3, b3):
    nf = jnp.concatenate([pos, t, jnp.zeros((N, 4), jnp.float32)],
                         axis=1)
    pad = jnp.arange(E_PAD - E, dtype=edge_index.dtype) % N
    ei = jnp.concatenate([edge_index[0], pad])
    ej = jnp.concatenate([edge_index[1], pad])
    parts = []
    svs = []
    eihs = []
    for h in range(2):
        eih = lax.slice(ei, (h * E_H,), ((h + 1) * E_H,))
        ejh = lax.slice(ej, (h * E_H,), ((h + 1) * E_H,))
        dt, dx, dy, dz = _gather(nf, eih, ejh)
        s2 = _mlp(dt.reshape(R128, 128), dx.reshape(R128, 128),
                  dy.reshape(R128, 128), dz.reshape(R128, 128),
                  W1, b1, W2, b2, W3, b3)
        parts.append(_scatter(s2.reshape(E_H), dx, dy, dz, eih))
    return _combine(parts[0], parts[1])
